# baseline (speedup 1.0000x reference)
import jax
import jax.numpy as jnp
from jax.experimental import pallas as pl
from jax.experimental.pallas import tpu as pltpu

_NUM_CLASSES = 8192
_Z_DIM = 256
_N_CHUNKS = 8
_CHUNK = _NUM_CLASSES // _N_CHUNKS


def _dma_body(a_ref, o_ref, sems):
    for i in range(_N_CHUNKS):
        rows = pl.ds(i * _CHUNK, _CHUNK)
        pltpu.make_async_copy(a_ref.at[rows, :], o_ref.at[rows, :], sems.at[i]).start()
    for i in range(_N_CHUNKS):
        rows = pl.ds(i * _CHUNK, _CHUNK)
        pltpu.make_async_copy(a_ref.at[rows, :], o_ref.at[rows, :], sems.at[i]).wait()


def kernel(_, anchor):
    return pl.pallas_call(
        _dma_body,
        in_specs=[pl.BlockSpec(memory_space=pl.ANY)],
        out_specs=pl.BlockSpec(memory_space=pl.ANY),
        out_shape=jax.ShapeDtypeStruct((_NUM_CLASSES, _Z_DIM), jnp.float32),
        scratch_shapes=[pltpu.SemaphoreType.DMA((_N_CHUNKS,))],
    )(anchor)


# speedup vs baseline: 20.1440x; 20.1440x over previous
import jax
import jax.numpy as jnp
from jax.experimental import pallas as pl
from jax.experimental.pallas import tpu as pltpu

_NUM_CLASSES = 8192
_Z_DIM = 256
_BLOCK_ROWS = 512


def _copy_body(a_ref, o_ref):
    o_ref[...] = a_ref[...]


def kernel(_, anchor):
    grid = (_NUM_CLASSES // _BLOCK_ROWS,)
    return pl.pallas_call(
        _copy_body,
        grid=grid,
        in_specs=[pl.BlockSpec((_BLOCK_ROWS, _Z_DIM), lambda i: (i, 0))],
        out_specs=pl.BlockSpec((_BLOCK_ROWS, _Z_DIM), lambda i: (i, 0)),
        out_shape=jax.ShapeDtypeStruct((_NUM_CLASSES, _Z_DIM), jnp.float32),
        compiler_params=pltpu.CompilerParams(dimension_semantics=("parallel",)),
    )(anchor)


# speedup vs baseline: 34.7814x; 1.7266x over previous
import jax
import jax.numpy as jnp
from jax.experimental import pallas as pl
from jax.experimental.pallas import tpu as pltpu

_NUM_CLASSES = 8192
_Z_DIM = 256
_BLOCK_ROWS = 2048


def _copy_body(a_ref, o_ref):
    o_ref[...] = a_ref[...]


def kernel(_, anchor):
    grid = (_NUM_CLASSES // _BLOCK_ROWS,)
    return pl.pallas_call(
        _copy_body,
        grid=grid,
        in_specs=[pl.BlockSpec((_BLOCK_ROWS, _Z_DIM), lambda i: (i, 0))],
        out_specs=pl.BlockSpec((_BLOCK_ROWS, _Z_DIM), lambda i: (i, 0)),
        out_shape=jax.ShapeDtypeStruct((_NUM_CLASSES, _Z_DIM), jnp.float32),
        compiler_params=pltpu.CompilerParams(dimension_semantics=("parallel",)),
    )(anchor)


# speedup vs baseline: 43.1705x; 1.2412x over previous
import jax
import jax.numpy as jnp
from jax.experimental import pallas as pl
from jax.experimental.pallas import tpu as pltpu

_NUM_CLASSES = 8192
_Z_DIM = 256
_BLOCK_ROWS = 4096


def _copy_body(a_ref, o_ref):
    o_ref[...] = a_ref[...]


def kernel(_, anchor):
    grid = (_NUM_CLASSES // _BLOCK_ROWS,)
    return pl.pallas_call(
        _copy_body,
        grid=grid,
        in_specs=[pl.BlockSpec((_BLOCK_ROWS, _Z_DIM), lambda i: (i, 0))],
        out_specs=pl.BlockSpec((_BLOCK_ROWS, _Z_DIM), lambda i: (i, 0)),
        out_shape=jax.ShapeDtypeStruct((_NUM_CLASSES, _Z_DIM), jnp.float32),
        compiler_params=pltpu.CompilerParams(dimension_semantics=("parallel",)),
    )(anchor)
